# baseline (speedup 1.0000x reference)
import dataclasses
import functools

import jax
import jax.numpy as jnp
from jax import lax
from jax.experimental import pallas as pl
from jax.experimental.pallas import tpu as pltpu
from jax.experimental.pallas import tpu_sc as plsc

N = 10000
D = 128
E = 320000
NC = 2
NS = 16
NW = NC * NS
EPW = E // NW
CB = 125
NCHUNK = EPW // CB
NPAD = 10240
RPT = NPAD // NS
DW = 16

_mesh = plsc.VectorSubcoreMesh(core_axis_name="c", subcore_axis_name="s")

_cp_no_layout = pltpu.CompilerParams()
if "needs_layout_passes" in pltpu.CompilerParams.__dataclass_fields__:
    _cp_no_layout = dataclasses.replace(_cp_no_layout, needs_layout_passes=False)


def _sc_scatter(y, srcs, dsts, zeros, H):
    @functools.partial(
        pl.kernel,
        out_type=jax.ShapeDtypeStruct((NC, NPAD, H), jnp.float32),
        mesh=_mesh,
        scratch_types=[
            pltpu.VMEM((NCHUNK // 2, CB), jnp.int32),
            pltpu.VMEM((NCHUNK // 2, CB), jnp.int32),
            pltpu.VMEM((CB, H), jnp.float32),
            pltpu.VMEM((CB, H), jnp.float32),
            pltpu.VMEM_SHARED((NPAD, H), jnp.float32),
            pltpu.SemaphoreType.DMA,
            pltpu.SemaphoreType.DMA,
            pltpu.SemaphoreType.DMA,
        ],
    )
    def k(y_hbm, srcs_hbm, dsts_hbm, zeros_hbm, dummy_hbm, out_hbm, src_v, dst_v,
          rows_a, rows_b, acc, sem_a, sem_b, sem_z):
        cid = lax.axis_index("c")
        sid = lax.axis_index("s")
        r0 = sid * RPT
        @pl.when(cid == 0)
        def _():
            pltpu.async_copy(y_hbm.at[pl.ds(r0, RPT)], acc.at[pl.ds(r0, RPT)], sem_z)

        @pl.when(cid == 1)
        def _():
            pltpu.async_copy(zeros_hbm.at[pl.ds(r0, RPT)], acc.at[pl.ds(r0, RPT)], sem_z)

        HCH = NCHUNK // 2
        for h in range(2):
            pltpu.sync_copy(srcs_hbm.at[cid, sid, pl.ds(h * HCH, HCH)], src_v)
            pltpu.sync_copy(dsts_hbm.at[cid, sid, pl.ds(h * HCH, HCH)], dst_v)
            if h == 0:
                pltpu.make_async_copy(zeros_hbm.at[pl.ds(r0, RPT)],
                                      acc.at[pl.ds(r0, RPT)], sem_z).wait()
                plsc.subcore_barrier()
            pltpu.async_copy(y_hbm.at[src_v.at[0]], rows_a, sem_a)

            @pl.loop(0, HCH, step=2)
            def _(j):
                pltpu.async_copy(y_hbm.at[src_v.at[j + 1]], rows_b, sem_b)
                pltpu.make_async_copy(dummy_hbm, rows_a, sem_a).wait()
                pltpu.sync_copy(rows_a, acc.at[dst_v.at[j]], add=True)

                @pl.when(j + 2 < HCH)
                def _():
                    pltpu.async_copy(y_hbm.at[src_v.at[j + 2]], rows_a, sem_a)

                pltpu.make_async_copy(dummy_hbm, rows_b, sem_b).wait()
                pltpu.sync_copy(rows_b, acc.at[dst_v.at[j + 1]], add=True)

        plsc.subcore_barrier()
        pltpu.sync_copy(acc.at[pl.ds(r0, RPT)], out_hbm.at[cid, pl.ds(r0, RPT)])

    return k(y, srcs, dsts, zeros, jnp.zeros((CB, H), jnp.float32))


def _sc_degree(dsts_flat, zflat):
    @functools.partial(
        pl.kernel,
        out_type=jax.ShapeDtypeStruct((NC, NS, RPT), jnp.float32),
        mesh=_mesh,
        scratch_types=[
            pltpu.VMEM((EPW,), jnp.int32),
            pltpu.VMEM((8 * NPAD,), jnp.float32),
            pltpu.VMEM((NS, RPT), jnp.float32),
            pltpu.VMEM((RPT,), jnp.float32),
            pltpu.VMEM_SHARED((NS, NPAD), jnp.float32),
        ],
        compiler_params=_cp_no_layout,
    )
    def k(dsts_hbm, zflat_hbm, out_hbm, dst_v, hist, red, res, shared):
        cid = lax.axis_index("c")
        sid = lax.axis_index("s")
        pltpu.sync_copy(dsts_hbm.at[cid, sid], dst_v)
        pltpu.sync_copy(zflat_hbm, hist)

        lane = lax.iota(jnp.int32, 16)
        colbase = (lane & 7) * NPAD
        ones16 = jnp.full((16,), 1.0, jnp.float32)
        m0 = lane < 8
        m1 = lane >= 8

        @pl.loop(0, EPW // 16)
        def _(i):
            idx = dst_v[pl.ds(i * 16, 16)] + colbase
            plsc.addupdate_scatter(hist, [idx], ones16, mask=m0)
            plsc.addupdate_scatter(hist, [idx], ones16, mask=m1)

        @pl.loop(0, NPAD // 16)
        def _(kk):
            s = hist[pl.ds(kk * 16, 16)]
            for c in range(1, 8):
                s = s + hist[pl.ds(c * NPAD + kk * 16, 16)]
            hist[pl.ds(kk * 16, 16)] = s

        pltpu.sync_copy(hist.at[pl.ds(0, NPAD)], shared.at[sid])
        plsc.subcore_barrier()
        for t in range(NS):
            pltpu.sync_copy(shared.at[t, pl.ds(sid * RPT, RPT)], red.at[t])

        @pl.loop(0, RPT // 16)
        def _(g):
            acc16 = red[0, pl.ds(g * 16, 16)]
            for t in range(1, NS):
                acc16 = acc16 + red[t, pl.ds(g * 16, 16)]
            res[pl.ds(g * 16, 16)] = acc16

        pltpu.sync_copy(res, out_hbm.at[cid, sid])

    return k(dsts_flat, zflat)


def _tc_matmul(x, W):
    def body(x_ref, w_ref, o_ref):
        o_ref[...] = jnp.dot(x_ref[...], w_ref[...],
                             preferred_element_type=jnp.float32)

    return pl.pallas_call(
        body,
        out_shape=jax.ShapeDtypeStruct((N, W.shape[1]), jnp.float32),
    )(x, W)


def _tc_prescale(y, degp):
    def body(y_ref, dg_ref, yp_ref, dinv_ref):
        deg = dg_ref[0][:N] + dg_ref[1][:N] + 1.0
        dinv = lax.rsqrt(deg)
        dinv_ref[...] = dinv
        yp = y_ref[...] * dinv
        yp_ref[...] = jnp.concatenate(
            [yp, jnp.zeros((NPAD - N, y_ref.shape[1]), jnp.float32)], axis=0)

    return pl.pallas_call(
        body,
        out_shape=(
            jax.ShapeDtypeStruct((NPAD, y.shape[1]), jnp.float32),
            jax.ShapeDtypeStruct((N, 1), jnp.float32),
        ),
    )(y, degp)


def _tc_stage(acc, dinv, b, g, be, Wn):
    Hn = Wn.shape[1]

    def body(acc_ref, dinv_ref, b_ref, g_ref, be_ref, w_ref, o_ref):
        dinv = dinv_ref[...]
        pre = (acc_ref[0][:N] + acc_ref[1][:N]) * dinv + b_ref[...]
        m = jnp.mean(pre, axis=0, keepdims=True)
        v = jnp.mean((pre - m) ** 2, axis=0, keepdims=True)
        h = jnp.maximum((pre - m) * lax.rsqrt(v + 1e-5) * g_ref[...] + be_ref[...], 0.0)
        yn = jnp.dot(h, w_ref[...], preferred_element_type=jnp.float32) * dinv
        o_ref[...] = jnp.concatenate(
            [yn, jnp.zeros((NPAD - N, Hn), jnp.float32)], axis=0)

    return pl.pallas_call(
        body,
        out_shape=jax.ShapeDtypeStruct((NPAD, Hn), jnp.float32),
    )(acc, dinv, b.reshape(1, -1), g.reshape(1, -1), be.reshape(1, -1), Wn)


def _tc_final(acc, dinv, b, g, be, Wc1, bc1, Wc2, bc2, Wv1, bv1, Wv2, bv2):
    def body(acc_ref, dinv_ref, b_ref, g_ref, be_ref,
             wc1_ref, bc1_ref, wc2_ref, bc2_ref, wv1_ref, bv1_ref, wv2_ref, bv2_ref,
             det_ref, vt_ref):
        pre = (acc_ref[0][:N, :64] + acc_ref[1][:N, :64]) * dinv_ref[...] + b_ref[...]
        m = jnp.mean(pre, axis=0, keepdims=True)
        v = jnp.mean((pre - m) ** 2, axis=0, keepdims=True)
        h = jnp.maximum((pre - m) * lax.rsqrt(v + 1e-5) * g_ref[...] + be_ref[...], 0.0)
        pooled = jnp.mean(h, axis=0, keepdims=True)
        dh = jnp.maximum(
            jnp.dot(pooled, wc1_ref[...], preferred_element_type=jnp.float32) + bc1_ref[...], 0.0)
        det_ref[...] = jnp.dot(dh, wc2_ref[...], preferred_element_type=jnp.float32) + bc2_ref[...]
        vh = jnp.maximum(
            jnp.dot(pooled, wv1_ref[...], preferred_element_type=jnp.float32) + bv1_ref[...], 0.0)
        vt_ref[...] = jnp.dot(vh, wv2_ref[...], preferred_element_type=jnp.float32) + bv2_ref[...]

    return pl.pallas_call(
        body,
        out_shape=(
            jax.ShapeDtypeStruct((1, Wc2.shape[1]), jnp.float32),
            jax.ShapeDtypeStruct((1, Wv2.shape[1]), jnp.float32),
        ),
    )(acc, dinv, b.reshape(1, -1), g.reshape(1, -1), be.reshape(1, -1),
      Wc1, bc1.reshape(1, -1), Wc2, bc2.reshape(1, -1),
      Wv1, bv1.reshape(1, -1), Wv2, bv2.reshape(1, -1))


def kernel(x, edge_index, W1, b1, W2, b2, W3, b3, g1, be1, g2, be2, g3, be3,
           Wc1, bc1, Wc2, bc2, Wv1, bv1, Wv2, bv2):
    src = edge_index[0].reshape(NC, NS, NCHUNK, CB)
    dst = edge_index[1].reshape(NC, NS, NCHUNK, CB)
    zeros128 = jnp.zeros((NPAD, 128), jnp.float32)
    dst_flat = edge_index[1].reshape(NC, NS, EPW)
    zflat = jnp.zeros((8 * NPAD,), jnp.float32)

    degp = _sc_degree(dst_flat, zflat).reshape(NC, NPAD, 1)
    y1 = _tc_matmul(x, W1)
    y1p, dinv = _tc_prescale(y1, degp)
    acc1 = _sc_scatter(y1p, src, dst, zeros128, 128)
    y2p = _tc_stage(acc1, dinv, b1, g1, be1, W2)
    acc2 = _sc_scatter(y2p, src, dst, zeros128, 128)
    W3p = jnp.pad(W3, ((0, 0), (0, 128 - W3.shape[1])))
    y3p = _tc_stage(acc2, dinv, b2, g2, be2, W3p)
    acc3 = _sc_scatter(y3p, src, dst, zeros128, 128)
    det, vt = _tc_final(acc3, dinv, b3, g3, be3,
                        Wc1, bc1, Wc2, bc2, Wv1, bv1, Wv2, bv2)
    return (det, vt)


# speedup vs baseline: 1.0019x; 1.0019x over previous
import dataclasses
import functools

import jax
import jax.numpy as jnp
from jax import lax
from jax.experimental import pallas as pl
from jax.experimental.pallas import tpu as pltpu
from jax.experimental.pallas import tpu_sc as plsc

N = 10000
D = 128
E = 320000
NC = 2
NS = 16
NW = NC * NS
EPW = E // NW
CB = 125
NCHUNK = EPW // CB
NPAD = 10240
RPT = NPAD // NS
DW = 16

_mesh = plsc.VectorSubcoreMesh(core_axis_name="c", subcore_axis_name="s")

_cp_no_layout = pltpu.CompilerParams()
if "needs_layout_passes" in pltpu.CompilerParams.__dataclass_fields__:
    _cp_no_layout = dataclasses.replace(_cp_no_layout, needs_layout_passes=False)


def _sc_scatter(y, srcs, dsts, zeros, H):
    @functools.partial(
        pl.kernel,
        out_type=jax.ShapeDtypeStruct((NC, NPAD, H), jnp.float32),
        mesh=_mesh,
        scratch_types=[
            pltpu.VMEM((NCHUNK // 2, CB), jnp.int32),
            pltpu.VMEM((NCHUNK // 2, CB), jnp.int32),
            pltpu.VMEM((CB, H), jnp.float32),
            pltpu.VMEM((CB, H), jnp.float32),
            pltpu.VMEM_SHARED((NPAD, H), jnp.float32),
            pltpu.SemaphoreType.DMA,
            pltpu.SemaphoreType.DMA,
            pltpu.SemaphoreType.DMA,
        ],
    )
    def k(y_hbm, srcs_hbm, dsts_hbm, zeros_hbm, dummy_hbm, out_hbm, src_v, dst_v,
          rows_a, rows_b, acc, sem_a, sem_b, sem_z):
        cid = lax.axis_index("c")
        sid = lax.axis_index("s")
        r0 = sid * RPT
        @pl.when(cid == 0)
        def _():
            pltpu.async_copy(y_hbm.at[pl.ds(r0, RPT)], acc.at[pl.ds(r0, RPT)], sem_z)

        @pl.when(cid == 1)
        def _():
            pltpu.async_copy(zeros_hbm.at[pl.ds(r0, RPT)], acc.at[pl.ds(r0, RPT)], sem_z)

        HCH = NCHUNK // 2
        for h in range(2):
            pltpu.sync_copy(srcs_hbm.at[cid, sid, pl.ds(h * HCH, HCH)], src_v)
            pltpu.sync_copy(dsts_hbm.at[cid, sid, pl.ds(h * HCH, HCH)], dst_v)
            if h == 0:
                pltpu.make_async_copy(zeros_hbm.at[pl.ds(r0, RPT)],
                                      acc.at[pl.ds(r0, RPT)], sem_z).wait()
                plsc.subcore_barrier()
            pltpu.async_copy(y_hbm.at[src_v.at[0]], rows_a, sem_a)

            @pl.loop(0, HCH, step=2)
            def _(j):
                pltpu.async_copy(y_hbm.at[src_v.at[j + 1]], rows_b, sem_b)
                pltpu.make_async_copy(dummy_hbm, rows_a, sem_a).wait()
                pltpu.sync_copy(rows_a, acc.at[dst_v.at[j]], add=True)

                @pl.when(j + 2 < HCH)
                def _():
                    pltpu.async_copy(y_hbm.at[src_v.at[j + 2]], rows_a, sem_a)

                pltpu.make_async_copy(dummy_hbm, rows_b, sem_b).wait()
                pltpu.sync_copy(rows_b, acc.at[dst_v.at[j + 1]], add=True)

        plsc.subcore_barrier()
        pltpu.sync_copy(acc.at[pl.ds(r0, RPT)], out_hbm.at[cid, pl.ds(r0, RPT)])

    return k(y, srcs, dsts, zeros, jnp.zeros((CB, H), jnp.float32))


def _sc_degree(dsts_flat, zflat):
    @functools.partial(
        pl.kernel,
        out_type=jax.ShapeDtypeStruct((NC, NS, RPT), jnp.float32),
        mesh=_mesh,
        scratch_types=[
            pltpu.VMEM((EPW,), jnp.int32),
            pltpu.VMEM((8 * NPAD,), jnp.float32),
            pltpu.VMEM((NS, RPT), jnp.float32),
            pltpu.VMEM((RPT,), jnp.float32),
            pltpu.VMEM_SHARED((NS, NPAD), jnp.float32),
        ],
        compiler_params=_cp_no_layout,
    )
    def k(dsts_hbm, zflat_hbm, out_hbm, dst_v, hist, red, res, shared):
        cid = lax.axis_index("c")
        sid = lax.axis_index("s")
        pltpu.sync_copy(dsts_hbm.at[cid, sid], dst_v)
        pltpu.sync_copy(zflat_hbm, hist)

        lane = lax.iota(jnp.int32, 16)
        colbase = (lane & 7) * NPAD
        ones16 = jnp.full((16,), 1.0, jnp.float32)
        m0 = lane < 8
        m1 = lane >= 8

        @pl.loop(0, EPW // 16)
        def _(i):
            idx = dst_v[pl.ds(i * 16, 16)] + colbase
            plsc.addupdate_scatter(hist, [idx], ones16, mask=m0)
            plsc.addupdate_scatter(hist, [idx], ones16, mask=m1)

        @pl.loop(0, NPAD // 16)
        def _(kk):
            s = hist[pl.ds(kk * 16, 16)]
            for c in range(1, 8):
                s = s + hist[pl.ds(c * NPAD + kk * 16, 16)]
            hist[pl.ds(kk * 16, 16)] = s

        pltpu.sync_copy(hist.at[pl.ds(0, NPAD)], shared.at[sid])
        plsc.subcore_barrier()
        for t in range(NS):
            pltpu.sync_copy(shared.at[t, pl.ds(sid * RPT, RPT)], red.at[t])

        @pl.loop(0, RPT // 16)
        def _(g):
            acc16 = red[0, pl.ds(g * 16, 16)]
            for t in range(1, NS):
                acc16 = acc16 + red[t, pl.ds(g * 16, 16)]
            res[pl.ds(g * 16, 16)] = acc16

        pltpu.sync_copy(res, out_hbm.at[cid, sid])

    return k(dsts_flat, zflat)


def _tc_prescale(x, W, degp):
    def body(x_ref, w_ref, dg_ref, yp_ref, dinv_ref):
        deg = dg_ref[0][:N] + dg_ref[1][:N] + 1.0
        dinv = lax.rsqrt(deg)
        dinv_ref[...] = dinv
        yp = jnp.dot(x_ref[...], w_ref[...],
                     preferred_element_type=jnp.float32) * dinv
        yp_ref[...] = jnp.concatenate(
            [yp, jnp.zeros((NPAD - N, W.shape[1]), jnp.float32)], axis=0)

    return pl.pallas_call(
        body,
        out_shape=(
            jax.ShapeDtypeStruct((NPAD, W.shape[1]), jnp.float32),
            jax.ShapeDtypeStruct((N, 1), jnp.float32),
        ),
    )(x, W, degp)


def _tc_stage(acc, dinv, b, g, be, Wn):
    Hn = Wn.shape[1]

    def body(acc_ref, dinv_ref, b_ref, g_ref, be_ref, w_ref, o_ref):
        dinv = dinv_ref[...]
        pre = (acc_ref[0][:N] + acc_ref[1][:N]) * dinv + b_ref[...]
        m = jnp.mean(pre, axis=0, keepdims=True)
        v = jnp.mean((pre - m) ** 2, axis=0, keepdims=True)
        h = jnp.maximum((pre - m) * lax.rsqrt(v + 1e-5) * g_ref[...] + be_ref[...], 0.0)
        yn = jnp.dot(h, w_ref[...], preferred_element_type=jnp.float32) * dinv
        o_ref[...] = jnp.concatenate(
            [yn, jnp.zeros((NPAD - N, Hn), jnp.float32)], axis=0)

    return pl.pallas_call(
        body,
        out_shape=jax.ShapeDtypeStruct((NPAD, Hn), jnp.float32),
    )(acc, dinv, b.reshape(1, -1), g.reshape(1, -1), be.reshape(1, -1), Wn)


def _tc_final(acc, dinv, b, g, be, Wc1, bc1, Wc2, bc2, Wv1, bv1, Wv2, bv2):
    def body(acc_ref, dinv_ref, b_ref, g_ref, be_ref,
             wc1_ref, bc1_ref, wc2_ref, bc2_ref, wv1_ref, bv1_ref, wv2_ref, bv2_ref,
             det_ref, vt_ref):
        pre = (acc_ref[0][:N, :64] + acc_ref[1][:N, :64]) * dinv_ref[...] + b_ref[...]
        m = jnp.mean(pre, axis=0, keepdims=True)
        v = jnp.mean((pre - m) ** 2, axis=0, keepdims=True)
        h = jnp.maximum((pre - m) * lax.rsqrt(v + 1e-5) * g_ref[...] + be_ref[...], 0.0)
        pooled = jnp.mean(h, axis=0, keepdims=True)
        dh = jnp.maximum(
            jnp.dot(pooled, wc1_ref[...], preferred_element_type=jnp.float32) + bc1_ref[...], 0.0)
        det_ref[...] = jnp.dot(dh, wc2_ref[...], preferred_element_type=jnp.float32) + bc2_ref[...]
        vh = jnp.maximum(
            jnp.dot(pooled, wv1_ref[...], preferred_element_type=jnp.float32) + bv1_ref[...], 0.0)
        vt_ref[...] = jnp.dot(vh, wv2_ref[...], preferred_element_type=jnp.float32) + bv2_ref[...]

    return pl.pallas_call(
        body,
        out_shape=(
            jax.ShapeDtypeStruct((1, Wc2.shape[1]), jnp.float32),
            jax.ShapeDtypeStruct((1, Wv2.shape[1]), jnp.float32),
        ),
    )(acc, dinv, b.reshape(1, -1), g.reshape(1, -1), be.reshape(1, -1),
      Wc1, bc1.reshape(1, -1), Wc2, bc2.reshape(1, -1),
      Wv1, bv1.reshape(1, -1), Wv2, bv2.reshape(1, -1))


def kernel(x, edge_index, W1, b1, W2, b2, W3, b3, g1, be1, g2, be2, g3, be3,
           Wc1, bc1, Wc2, bc2, Wv1, bv1, Wv2, bv2):
    src = edge_index[0].reshape(NC, NS, NCHUNK, CB)
    dst = edge_index[1].reshape(NC, NS, NCHUNK, CB)
    zeros128 = jnp.zeros((NPAD, 128), jnp.float32)
    dst_flat = edge_index[1].reshape(NC, NS, EPW)
    zflat = jnp.zeros((8 * NPAD,), jnp.float32)

    degp = _sc_degree(dst_flat, zflat).reshape(NC, NPAD, 1)
    y1p, dinv = _tc_prescale(x, W1, degp)
    acc1 = _sc_scatter(y1p, src, dst, zeros128, 128)
    y2p = _tc_stage(acc1, dinv, b1, g1, be1, W2)
    acc2 = _sc_scatter(y2p, src, dst, zeros128, 128)
    W3p = jnp.pad(W3, ((0, 0), (0, 128 - W3.shape[1])))
    y3p = _tc_stage(acc2, dinv, b2, g2, be2, W3p)
    acc3 = _sc_scatter(y3p, src, dst, zeros128, 128)
    det, vt = _tc_final(acc3, dinv, b3, g3, be3,
                        Wc1, bc1, Wc2, bc2, Wv1, bv1, Wv2, bv2)
    return (det, vt)


# speedup vs baseline: 1.0173x; 1.0154x over previous
import dataclasses
import functools

import jax
import jax.numpy as jnp
from jax import lax
from jax.experimental import pallas as pl
from jax.experimental.pallas import tpu as pltpu
from jax.experimental.pallas import tpu_sc as plsc

N = 10000
D = 128
E = 320000
NC = 2
NS = 16
NW = NC * NS
EPW = E // NW
CB = 125
NCHUNK = EPW // CB
NPAD = 10240
RPT = NPAD // NS
DW = 16

_mesh = plsc.VectorSubcoreMesh(core_axis_name="c", subcore_axis_name="s")

_cp_no_layout = pltpu.CompilerParams()
if "needs_layout_passes" in pltpu.CompilerParams.__dataclass_fields__:
    _cp_no_layout = dataclasses.replace(_cp_no_layout, needs_layout_passes=False)


def _sc_scatter(y, srcs, dsts, zeros, H):
    @functools.partial(
        pl.kernel,
        out_type=jax.ShapeDtypeStruct((NC, NPAD, H), jnp.float32),
        mesh=_mesh,
        scratch_types=[
            pltpu.VMEM((NCHUNK // 2, CB), jnp.int32),
            pltpu.VMEM((NCHUNK // 2, CB), jnp.int32),
            pltpu.VMEM((CB, H), jnp.float32),
            pltpu.VMEM((CB, H), jnp.float32),
            pltpu.VMEM_SHARED((NPAD, H), jnp.float32),
            pltpu.SemaphoreType.DMA,
            pltpu.SemaphoreType.DMA,
            pltpu.SemaphoreType.DMA,
        ],
    )
    def k(y_hbm, srcs_hbm, dsts_hbm, zeros_hbm, dummy_hbm, out_hbm, src_v, dst_v,
          rows_a, rows_b, acc, sem_a, sem_b, sem_z):
        cid = lax.axis_index("c")
        sid = lax.axis_index("s")
        r0 = sid * RPT
        @pl.when(cid == 0)
        def _():
            pltpu.async_copy(y_hbm.at[pl.ds(r0, RPT)], acc.at[pl.ds(r0, RPT)], sem_z)

        @pl.when(cid == 1)
        def _():
            pltpu.async_copy(zeros_hbm.at[pl.ds(r0, RPT)], acc.at[pl.ds(r0, RPT)], sem_z)

        HCH = NCHUNK // 2
        for h in range(2):
            pltpu.sync_copy(srcs_hbm.at[cid, sid, pl.ds(h * HCH, HCH)], src_v)
            pltpu.sync_copy(dsts_hbm.at[cid, sid, pl.ds(h * HCH, HCH)], dst_v)
            if h == 0:
                pltpu.make_async_copy(zeros_hbm.at[pl.ds(r0, RPT)],
                                      acc.at[pl.ds(r0, RPT)], sem_z).wait()
                plsc.subcore_barrier()
            pltpu.async_copy(y_hbm.at[src_v.at[0]], rows_a, sem_a)

            @pl.loop(0, HCH, step=2)
            def _(j):
                pltpu.async_copy(y_hbm.at[src_v.at[j + 1]], rows_b, sem_b)
                pltpu.make_async_copy(dummy_hbm, rows_a, sem_a).wait()
                pltpu.sync_copy(rows_a, acc.at[dst_v.at[j]], add=True)

                @pl.when(j + 2 < HCH)
                def _():
                    pltpu.async_copy(y_hbm.at[src_v.at[j + 2]], rows_a, sem_a)

                pltpu.make_async_copy(dummy_hbm, rows_b, sem_b).wait()
                pltpu.sync_copy(rows_b, acc.at[dst_v.at[j + 1]], add=True)

        plsc.subcore_barrier()
        pltpu.sync_copy(acc.at[pl.ds(r0, RPT)], out_hbm.at[cid, pl.ds(r0, RPT)])

    return k(y, srcs, dsts, zeros, jnp.zeros((CB, H), jnp.float32))


def _sc_degree(dsts_flat, zflat):
    @functools.partial(
        pl.kernel,
        out_type=jax.ShapeDtypeStruct((NC, NS, RPT), jnp.float32),
        mesh=_mesh,
        scratch_types=[
            pltpu.VMEM((EPW,), jnp.int32),
            pltpu.VMEM((4 * NPAD,), jnp.float32),
            pltpu.VMEM((NS, RPT), jnp.float32),
            pltpu.VMEM((RPT,), jnp.float32),
            pltpu.VMEM_SHARED((NS, NPAD), jnp.float32),
        ],
        compiler_params=_cp_no_layout,
    )
    def k(dsts_hbm, zflat_hbm, out_hbm, dst_v, hist, red, res, shared):
        cid = lax.axis_index("c")
        sid = lax.axis_index("s")
        pltpu.sync_copy(dsts_hbm.at[cid, sid], dst_v)
        pltpu.sync_copy(zflat_hbm, hist)

        lane = lax.iota(jnp.int32, 16)
        colbase = (lane & 3) * NPAD
        ones16 = jnp.full((16,), 1.0, jnp.float32)
        m0 = lane < 4
        m1 = (lane >= 4) & (lane < 8)
        m2 = (lane >= 8) & (lane < 12)
        m3 = lane >= 12

        @pl.loop(0, EPW // 16)
        def _(i):
            idx = dst_v[pl.ds(i * 16, 16)] + colbase
            plsc.addupdate_scatter(hist, [idx], ones16, mask=m0)
            plsc.addupdate_scatter(hist, [idx], ones16, mask=m1)
            plsc.addupdate_scatter(hist, [idx], ones16, mask=m2)
            plsc.addupdate_scatter(hist, [idx], ones16, mask=m3)

        @pl.loop(0, NPAD // 16)
        def _(kk):
            s = (hist[pl.ds(kk * 16, 16)]
                 + hist[pl.ds(NPAD + kk * 16, 16)]
                 + hist[pl.ds(2 * NPAD + kk * 16, 16)]
                 + hist[pl.ds(3 * NPAD + kk * 16, 16)])
            hist[pl.ds(kk * 16, 16)] = s

        pltpu.sync_copy(hist.at[pl.ds(0, NPAD)], shared.at[sid])
        plsc.subcore_barrier()
        for t in range(NS):
            pltpu.sync_copy(shared.at[t, pl.ds(sid * RPT, RPT)], red.at[t])

        @pl.loop(0, RPT // 16)
        def _(g):
            acc16 = red[0, pl.ds(g * 16, 16)]
            for t in range(1, NS):
                acc16 = acc16 + red[t, pl.ds(g * 16, 16)]
            res[pl.ds(g * 16, 16)] = acc16

        pltpu.sync_copy(res, out_hbm.at[cid, sid])

    return k(dsts_flat, zflat)


def _tc_prescale(x, W, degp):
    def body(x_ref, w_ref, dg_ref, yp_ref, dinv_ref):
        deg = dg_ref[0][:N] + dg_ref[1][:N] + 1.0
        dinv = lax.rsqrt(deg)
        dinv_ref[...] = dinv
        yp = jnp.dot(x_ref[...], w_ref[...],
                     preferred_element_type=jnp.float32) * dinv
        yp_ref[...] = jnp.concatenate(
            [yp, jnp.zeros((NPAD - N, W.shape[1]), jnp.float32)], axis=0)

    return pl.pallas_call(
        body,
        out_shape=(
            jax.ShapeDtypeStruct((NPAD, W.shape[1]), jnp.float32),
            jax.ShapeDtypeStruct((N, 1), jnp.float32),
        ),
    )(x, W, degp)


def _tc_stage(acc, dinv, b, g, be, Wn):
    Hn = Wn.shape[1]

    def body(acc_ref, dinv_ref, b_ref, g_ref, be_ref, w_ref, o_ref):
        dinv = dinv_ref[...]
        pre = (acc_ref[0][:N] + acc_ref[1][:N]) * dinv + b_ref[...]
        m = jnp.mean(pre, axis=0, keepdims=True)
        v = jnp.mean((pre - m) ** 2, axis=0, keepdims=True)
        h = jnp.maximum((pre - m) * lax.rsqrt(v + 1e-5) * g_ref[...] + be_ref[...], 0.0)
        yn = jnp.dot(h, w_ref[...], preferred_element_type=jnp.float32) * dinv
        o_ref[...] = jnp.concatenate(
            [yn, jnp.zeros((NPAD - N, Hn), jnp.float32)], axis=0)

    return pl.pallas_call(
        body,
        out_shape=jax.ShapeDtypeStruct((NPAD, Hn), jnp.float32),
    )(acc, dinv, b.reshape(1, -1), g.reshape(1, -1), be.reshape(1, -1), Wn)


def _tc_final(acc, dinv, b, g, be, Wc1, bc1, Wc2, bc2, Wv1, bv1, Wv2, bv2):
    def body(acc_ref, dinv_ref, b_ref, g_ref, be_ref,
             wc1_ref, bc1_ref, wc2_ref, bc2_ref, wv1_ref, bv1_ref, wv2_ref, bv2_ref,
             det_ref, vt_ref):
        pre = (acc_ref[0][:N, :64] + acc_ref[1][:N, :64]) * dinv_ref[...] + b_ref[...]
        m = jnp.mean(pre, axis=0, keepdims=True)
        v = jnp.mean((pre - m) ** 2, axis=0, keepdims=True)
        h = jnp.maximum((pre - m) * lax.rsqrt(v + 1e-5) * g_ref[...] + be_ref[...], 0.0)
        pooled = jnp.mean(h, axis=0, keepdims=True)
        dh = jnp.maximum(
            jnp.dot(pooled, wc1_ref[...], preferred_element_type=jnp.float32) + bc1_ref[...], 0.0)
        det_ref[...] = jnp.dot(dh, wc2_ref[...], preferred_element_type=jnp.float32) + bc2_ref[...]
        vh = jnp.maximum(
            jnp.dot(pooled, wv1_ref[...], preferred_element_type=jnp.float32) + bv1_ref[...], 0.0)
        vt_ref[...] = jnp.dot(vh, wv2_ref[...], preferred_element_type=jnp.float32) + bv2_ref[...]

    return pl.pallas_call(
        body,
        out_shape=(
            jax.ShapeDtypeStruct((1, Wc2.shape[1]), jnp.float32),
            jax.ShapeDtypeStruct((1, Wv2.shape[1]), jnp.float32),
        ),
    )(acc, dinv, b.reshape(1, -1), g.reshape(1, -1), be.reshape(1, -1),
      Wc1, bc1.reshape(1, -1), Wc2, bc2.reshape(1, -1),
      Wv1, bv1.reshape(1, -1), Wv2, bv2.reshape(1, -1))


def kernel(x, edge_index, W1, b1, W2, b2, W3, b3, g1, be1, g2, be2, g3, be3,
           Wc1, bc1, Wc2, bc2, Wv1, bv1, Wv2, bv2):
    src = edge_index[0].reshape(NC, NS, NCHUNK, CB)
    dst = edge_index[1].reshape(NC, NS, NCHUNK, CB)
    zeros128 = jnp.zeros((NPAD, 128), jnp.float32)
    dst_flat = edge_index[1].reshape(NC, NS, EPW)
    zflat = jnp.zeros((4 * NPAD,), jnp.float32)

    degp = _sc_degree(dst_flat, zflat).reshape(NC, NPAD, 1)
    y1p, dinv = _tc_prescale(x, W1, degp)
    acc1 = _sc_scatter(y1p, src, dst, zeros128, 128)
    y2p = _tc_stage(acc1, dinv, b1, g1, be1, W2)
    acc2 = _sc_scatter(y2p, src, dst, zeros128, 128)
    W3p = jnp.pad(W3, ((0, 0), (0, 128 - W3.shape[1])))
    y3p = _tc_stage(acc2, dinv, b2, g2, be2, W3p)
    acc3 = _sc_scatter(y3p, src, dst, zeros128, 128)
    det, vt = _tc_final(acc3, dinv, b3, g3, be3,
                        Wc1, bc1, Wc2, bc2, Wv1, bv1, Wv2, bv2)
    return (det, vt)
